# baseline (speedup 1.0000x reference)
import jax
import jax.numpy as jnp
from jax import lax
from jax.experimental import pallas as pl
from jax.experimental.pallas import tpu as pltpu
from jax.experimental.pallas import tpu_sc as plsc

_DIM = 64
_PAD = 128
_SCALE = 8.0
_NW = 32
_L = 16


def _mesh():
    return plsc.VectorSubcoreMesh(core_axis_name="c", subcore_axis_name="s")


def _wid():
    return lax.axis_index("s") * 2 + lax.axis_index("c")


def _transpose_scale(t_t, tail_p):
    d, v = t_t.shape
    chunk = 256
    n_full = v // chunk
    v_tail = n_full * chunk
    tail_n = v - v_tail

    @pl.kernel(
        out_type=jax.ShapeDtypeStruct((v, _PAD), jnp.float32),
        mesh=_mesh(),
        scratch_types=[
            [pltpu.VMEM((d, chunk), jnp.float32) for _ in range(2)],
            [pltpu.VMEM((chunk, _PAD), jnp.float32) for _ in range(2)],
            [pltpu.SemaphoreType.DMA for _ in range(2)],
            [pltpu.SemaphoreType.DMA for _ in range(2)],
        ],
        compiler_params=pltpu.CompilerParams(use_tc_tiling_on_sc=True, needs_layout_passes=False),
    )
    def k(t_hbm, tail_hbm, out_hbm, ibufs, obufs, isems, osems):
        w = _wid()
        iota = lax.iota(jnp.int32, _L)

        def start_in(c, b):
            pltpu.async_copy(
                t_hbm.at[:, pl.ds(c * chunk, chunk)], ibufs[b], isems[b]
            )

        def wait_in(b):
            pltpu.make_async_copy(
                t_hbm.at[:, pl.ds(0, chunk)], ibufs[b], isems[b]
            ).wait()

        def start_out(c, b):
            pltpu.async_copy(
                obufs[b], out_hbm.at[pl.ds(c * chunk, chunk)], osems[b]
            )

        def wait_out(b):
            pltpu.make_async_copy(
                obufs[b], out_hbm.at[pl.ds(0, chunk)], osems[b]
            ).wait()

        row_idx = [iota + (m * _L) for m in range(_DIM // _L)]

        def transpose(b):
            def col(r, carry):
                rv = jnp.full((_L,), 0, jnp.int32) + r
                for m in range(_DIM // _L):
                    vals = plsc.load_gather(ibufs[b], [row_idx[m], rv])
                    obufs[b][r, pl.ds(m * _L, _L)] = vals * _SCALE
                return carry

            lax.fori_loop(0, chunk, col, 0, unroll=8)

        n_mine = (n_full - w + _NW - 1) // _NW

        def step(t, carry):
            c = w + t * _NW
            b = lax.rem(t, 2)

            def do(bb):
                @pl.when(b == bb)
                def _():
                    @pl.when(t >= 2)
                    def _():
                        wait_out(bb)

                    wait_in(bb)
                    transpose(bb)
                    start_out(c, bb)

                    @pl.when(t + 2 < n_mine)
                    def _():
                        start_in(c + 2 * _NW, bb)

            do(0)
            do(1)
            return carry

        @pl.when(n_mine >= 1)
        def _():
            start_in(w, 0)

        @pl.when(n_mine >= 2)
        def _():
            start_in(w + _NW, 1)

        lax.fori_loop(0, n_mine, step, 0)

        @pl.when(n_mine >= 1)
        def _():
            wait_out(0)

        @pl.when(n_mine >= 2)
        def _():
            wait_out(1)

        @pl.when(w == 0)
        def _():
            if tail_n > 0:
                pltpu.sync_copy(tail_hbm, ibufs[0].at[:, pl.ds(0, _PAD)])

                def tcol(r, carry):
                    rv = jnp.full((_L,), 0, jnp.int32) + r
                    for m in range(_DIM // _L):
                        vals = plsc.load_gather(ibufs[0], [row_idx[m], rv])
                        obufs[0][r, pl.ds(m * _L, _L)] = vals * _SCALE
                    return carry

                lax.fori_loop(0, tail_n, tcol, 0)
                pltpu.sync_copy(
                    obufs[0].at[pl.ds(0, tail_n)], out_hbm.at[pl.ds(v_tail, tail_n)]
                )

    return k(t_t, tail_p)


def _gather_t(x_t, tl):
    s, n = x_t.shape
    cols_per_w = n // _NW
    blk = 128
    nblk = cols_per_w // blk

    @pl.kernel(
        out_type=jax.ShapeDtypeStruct((s, _DIM, n), jnp.float32),
        mesh=_mesh(),
        scratch_types=[
            pltpu.VMEM((s, cols_per_w), jnp.int32),
            [pltpu.VMEM((blk, _PAD), jnp.float32) for _ in range(2)],
            [pltpu.VMEM((_DIM, blk), jnp.float32) for _ in range(2)],
            [pltpu.SemaphoreType.DMA for _ in range(2)],
            [pltpu.SemaphoreType.DMA for _ in range(2)],
        ],
        compiler_params=pltpu.CompilerParams(use_tc_tiling_on_sc=True, needs_layout_passes=False),
    )
    def k(x_hbm, tl_hbm, out_hbm, idx_v, gbufs, obufs, gsems, osems):
        w = _wid()
        i0 = w * cols_per_w
        iota = lax.iota(jnp.int32, _L)

        def srow(j, carry):
            pltpu.sync_copy(x_hbm.at[j, pl.ds(i0, cols_per_w)], idx_v.at[j])
            return carry

        lax.fori_loop(0, s, srow, 0)

        nsteps = s * nblk

        def start_gather(t, b):
            j = t // nblk
            ib = lax.rem(t, nblk)
            pltpu.async_copy(
                tl_hbm.at[idx_v.at[j, pl.ds(ib * blk, blk)]], gbufs[b], gsems[b]
            )

        def wait_gather(b):
            pltpu.make_async_copy(
                tl_hbm.at[idx_v.at[0, pl.ds(0, blk)]], gbufs[b], gsems[b]
            ).wait()

        def start_out(t, b):
            j = t // nblk
            ib = lax.rem(t, nblk)
            pltpu.async_copy(
                obufs[b],
                out_hbm.at[j, :, pl.ds(i0 + ib * blk, blk)],
                osems[b],
            )

        def wait_out(b):
            pltpu.make_async_copy(
                obufs[b], out_hbm.at[0, :, pl.ds(0, blk)], osems[b]
            ).wait()

        row_idx = [iota + (m * _L) for m in range(blk // _L)]

        def transpose(b):
            def dloop(dd, carry):
                dv = jnp.full((_L,), 0, jnp.int32) + dd
                for m in range(blk // _L):
                    vals = plsc.load_gather(gbufs[b], [row_idx[m], dv])
                    obufs[b][dd, pl.ds(m * _L, _L)] = vals
                return carry

            lax.fori_loop(0, _DIM, dloop, 0, unroll=8)

        def step(t, carry):
            b = lax.rem(t, 2)

            def do(bb):
                @pl.when(b == bb)
                def _():
                    @pl.when(t >= 2)
                    def _():
                        wait_out(bb)

                    wait_gather(bb)
                    transpose(bb)
                    start_out(t, bb)

                    @pl.when(t + 2 < nsteps)
                    def _():
                        start_gather(t + 2, bb)

            do(0)
            do(1)
            return carry

        start_gather(0, 0)
        start_gather(1, 1)
        lax.fori_loop(0, nsteps, step, 0)
        wait_out(0)
        wait_out(1)

    return k(x_t, tl)


def kernel(x, table):
    n, s = x.shape
    v, d = table.shape
    chunk = 256
    v_tail = (v // chunk) * chunk
    tail_n = v - v_tail
    t_t = jnp.transpose(table)
    tail_p = jnp.pad(
        jnp.transpose(lax.slice(table, (v_tail, 0), (v, d))),
        ((0, 0), (0, _PAD - tail_n)),
    )
    tl = _transpose_scale(t_t, tail_p)
    x_t = jnp.transpose(x).astype(jnp.int32)
    y = _gather_t(x_t, tl)
    return jnp.transpose(y, (2, 0, 1))


# speedup vs baseline: 1.0004x; 1.0004x over previous
import jax
import jax.numpy as jnp
from jax import lax
from jax.experimental import pallas as pl
from jax.experimental.pallas import tpu as pltpu
from jax.experimental.pallas import tpu_sc as plsc

_DIM = 64
_PAD = 128
_SCALE = 8.0
_NW = 32
_L = 16


def _mesh():
    return plsc.VectorSubcoreMesh(core_axis_name="c", subcore_axis_name="s")


def _wid():
    return lax.axis_index("s") * 2 + lax.axis_index("c")


def _transpose_scale(t_t, tail_p):
    d, v = t_t.shape
    chunk = 256
    n_full = v // chunk
    v_tail = n_full * chunk
    tail_n = v - v_tail

    @pl.kernel(
        out_type=jax.ShapeDtypeStruct((v, _PAD), jnp.float32),
        mesh=_mesh(),
        scratch_types=[
            [pltpu.VMEM((d, chunk), jnp.float32) for _ in range(2)],
            [pltpu.VMEM((chunk, _PAD), jnp.float32) for _ in range(2)],
            [pltpu.SemaphoreType.DMA for _ in range(2)],
            [pltpu.SemaphoreType.DMA for _ in range(2)],
        ],
        compiler_params=pltpu.CompilerParams(use_tc_tiling_on_sc=True, needs_layout_passes=False),
    )
    def k(t_hbm, tail_hbm, out_hbm, ibufs, obufs, isems, osems):
        w = _wid()
        iota = lax.iota(jnp.int32, _L)

        def start_in(c, b):
            pltpu.async_copy(
                t_hbm.at[:, pl.ds(c * chunk, chunk)], ibufs[b], isems[b]
            )

        def wait_in(b):
            pltpu.make_async_copy(
                t_hbm.at[:, pl.ds(0, chunk)], ibufs[b], isems[b]
            ).wait()

        def start_out(c, b):
            pltpu.async_copy(
                obufs[b], out_hbm.at[pl.ds(c * chunk, chunk)], osems[b]
            )

        def wait_out(b):
            pltpu.make_async_copy(
                obufs[b], out_hbm.at[pl.ds(0, chunk)], osems[b]
            ).wait()

        row_idx = [iota + (m * _L) for m in range(_DIM // _L)]

        def transpose(b):
            unr = 8

            def col(t, carry):
                r0 = t * unr
                for u in range(unr):
                    r = r0 + u
                    rv = jnp.full((_L,), 0, jnp.int32) + r
                    for m in range(_DIM // _L):
                        vals = plsc.load_gather(ibufs[b], [row_idx[m], rv])
                        obufs[b][r, pl.ds(m * _L, _L)] = vals * _SCALE
                return carry

            lax.fori_loop(0, chunk // unr, col, 0)

        n_mine = (n_full - w + _NW - 1) // _NW

        def step(t, carry):
            c = w + t * _NW
            b = lax.rem(t, 2)

            def do(bb):
                @pl.when(b == bb)
                def _():
                    @pl.when(t >= 2)
                    def _():
                        wait_out(bb)

                    wait_in(bb)
                    transpose(bb)
                    start_out(c, bb)

                    @pl.when(t + 2 < n_mine)
                    def _():
                        start_in(c + 2 * _NW, bb)

            do(0)
            do(1)
            return carry

        @pl.when(n_mine >= 1)
        def _():
            start_in(w, 0)

        @pl.when(n_mine >= 2)
        def _():
            start_in(w + _NW, 1)

        lax.fori_loop(0, n_mine, step, 0)

        @pl.when(n_mine >= 1)
        def _():
            wait_out(0)

        @pl.when(n_mine >= 2)
        def _():
            wait_out(1)

        @pl.when(w == 0)
        def _():
            if tail_n > 0:
                pltpu.sync_copy(tail_hbm, ibufs[0].at[:, pl.ds(0, _PAD)])

                def tcol(r, carry):
                    rv = jnp.full((_L,), 0, jnp.int32) + r
                    for m in range(_DIM // _L):
                        vals = plsc.load_gather(ibufs[0], [row_idx[m], rv])
                        obufs[0][r, pl.ds(m * _L, _L)] = vals * _SCALE
                    return carry

                lax.fori_loop(0, tail_n, tcol, 0)
                pltpu.sync_copy(
                    obufs[0].at[pl.ds(0, tail_n)], out_hbm.at[pl.ds(v_tail, tail_n)]
                )

    return k(t_t, tail_p)


def _gather_t(x_t, tl):
    s, n = x_t.shape
    cols_per_w = n // _NW
    blk = 128
    nblk = cols_per_w // blk

    @pl.kernel(
        out_type=jax.ShapeDtypeStruct((s, _DIM, n), jnp.float32),
        mesh=_mesh(),
        scratch_types=[
            pltpu.VMEM((s, cols_per_w), jnp.int32),
            [pltpu.VMEM((blk, _PAD), jnp.float32) for _ in range(2)],
            [pltpu.VMEM((_DIM, blk), jnp.float32) for _ in range(2)],
            [pltpu.SemaphoreType.DMA for _ in range(2)],
            [pltpu.SemaphoreType.DMA for _ in range(2)],
        ],
        compiler_params=pltpu.CompilerParams(use_tc_tiling_on_sc=True, needs_layout_passes=False),
    )
    def k(x_hbm, tl_hbm, out_hbm, idx_v, gbufs, obufs, gsems, osems):
        w = _wid()
        i0 = w * cols_per_w
        iota = lax.iota(jnp.int32, _L)

        def srow(j, carry):
            pltpu.sync_copy(x_hbm.at[j, pl.ds(i0, cols_per_w)], idx_v.at[j])
            return carry

        lax.fori_loop(0, s, srow, 0)

        nsteps = s * nblk

        def start_gather(t, b):
            j = t // nblk
            ib = lax.rem(t, nblk)
            pltpu.async_copy(
                tl_hbm.at[idx_v.at[j, pl.ds(ib * blk, blk)]], gbufs[b], gsems[b]
            )

        def wait_gather(b):
            pltpu.make_async_copy(
                tl_hbm.at[idx_v.at[0, pl.ds(0, blk)]], gbufs[b], gsems[b]
            ).wait()

        def start_out(t, b):
            j = t // nblk
            ib = lax.rem(t, nblk)
            pltpu.async_copy(
                obufs[b],
                out_hbm.at[j, :, pl.ds(i0 + ib * blk, blk)],
                osems[b],
            )

        def wait_out(b):
            pltpu.make_async_copy(
                obufs[b], out_hbm.at[0, :, pl.ds(0, blk)], osems[b]
            ).wait()

        row_idx = [iota + (m * _L) for m in range(blk // _L)]

        def transpose(b):
            unr = 4

            def dloop(t, carry):
                d0 = t * unr
                for u in range(unr):
                    dd = d0 + u
                    dv = jnp.full((_L,), 0, jnp.int32) + dd
                    for m in range(blk // _L):
                        vals = plsc.load_gather(gbufs[b], [row_idx[m], dv])
                        obufs[b][dd, pl.ds(m * _L, _L)] = vals
                return carry

            lax.fori_loop(0, _DIM // unr, dloop, 0)

        def step(t, carry):
            b = lax.rem(t, 2)

            def do(bb):
                @pl.when(b == bb)
                def _():
                    @pl.when(t >= 2)
                    def _():
                        wait_out(bb)

                    wait_gather(bb)
                    transpose(bb)
                    start_out(t, bb)

                    @pl.when(t + 2 < nsteps)
                    def _():
                        start_gather(t + 2, bb)

            do(0)
            do(1)
            return carry

        start_gather(0, 0)
        start_gather(1, 1)
        lax.fori_loop(0, nsteps, step, 0)
        wait_out(0)
        wait_out(1)

    return k(x_t, tl)


def kernel(x, table):
    n, s = x.shape
    v, d = table.shape
    chunk = 256
    v_tail = (v // chunk) * chunk
    tail_n = v - v_tail
    t_t = jnp.transpose(table)
    tail_p = jnp.pad(
        jnp.transpose(lax.slice(table, (v_tail, 0), (v, d))),
        ((0, 0), (0, _PAD - tail_n)),
    )
    tl = _transpose_scale(t_t, tail_p)
    x_t = jnp.transpose(x).astype(jnp.int32)
    y = _gather_t(x_t, tl)
    return jnp.transpose(y, (2, 0, 1))


# speedup vs baseline: 2.0467x; 2.0458x over previous
import jax
import jax.numpy as jnp
from jax import lax
from jax.experimental import pallas as pl
from jax.experimental.pallas import tpu as pltpu
from jax.experimental.pallas import tpu_sc as plsc

_DIM = 64
_SCALE = 8.0
_NW = 32


def _sc_embed(x, table):
    n, s = x.shape
    rows_per_w = n // _NW

    mesh = plsc.VectorSubcoreMesh(core_axis_name="c", subcore_axis_name="s")

    @pl.kernel(
        out_type=jax.ShapeDtypeStruct((n, s, _DIM), jnp.float32),
        mesh=mesh,
        scratch_types=[
            pltpu.VMEM((rows_per_w, s), jnp.int32),
            [pltpu.VMEM((s, _DIM), jnp.float32) for _ in range(2)],
            [pltpu.SemaphoreType.DMA for _ in range(2)],
        ],
        compiler_params=pltpu.CompilerParams(
            use_tc_tiling_on_sc=False, needs_layout_passes=False
        ),
    )
    def k(x_hbm, table_hbm, out_hbm, idx_v, bufs, sems):
        wid = lax.axis_index("s") * 2 + lax.axis_index("c")
        i0 = wid * rows_per_w
        pltpu.sync_copy(x_hbm.at[pl.ds(i0, rows_per_w)], idx_v)

        def start_gather(j, b):
            pltpu.async_copy(table_hbm.at[idx_v.at[j]], bufs[b], sems[b])

        def wait_gather(b):
            pltpu.make_async_copy(
                table_hbm.at[idx_v.at[0]], bufs[b], sems[b]
            ).wait()

        def process(j, b):
            wait_gather(b)

            def srow(r, c):
                for kk in range(_DIM // 16):
                    sl = pl.ds(kk * 16, 16)
                    bufs[b][r, sl] = bufs[b][r, sl] * _SCALE
                return c

            lax.fori_loop(0, s, srow, 0, unroll=2)
            pltpu.sync_copy(bufs[b], out_hbm.at[i0 + j])

        start_gather(0, 0)

        def pair(g, c):
            j = g * 2
            start_gather(j + 1, 1)
            process(j, 0)
            start_gather(j + 2, 0)
            process(j + 1, 1)
            return c

        lax.fori_loop(0, rows_per_w // 2 - 1, pair, 0)

        j_last = rows_per_w - 2
        start_gather(j_last + 1, 1)
        process(j_last, 0)
        process(j_last + 1, 1)

    return k(x, table)


def kernel(x, table):
    return _sc_embed(x.astype(jnp.int32), table)
